# Initial kernel scaffold; baseline (speedup 1.0000x reference)
import jax
import jax.numpy as jnp
from jax.experimental import pallas as pl


def kernel(loc, conf, target_boxes, target_labels):
    raise NotImplementedError("write your pallas kernel here")


# speedup vs baseline: 17.5753x; 17.5753x over previous
import jax
import jax.numpy as jnp
from jax.experimental import pallas as pl
from jax.experimental.pallas import tpu as pltpu

_B = 128
_IOU_T = 0.5


def _nms_kernel(x1_ref, y1_ref, x2_ref, y2_ref, keep_ref, area_ref):
    nb = x1_ref.shape[0]
    area_ref[...] = (x2_ref[...] - x1_ref[...]) * (y2_ref[...] - y1_ref[...])
    keep_ref[...] = jnp.ones((nb, _B), jnp.int32)

    row_ids = jax.lax.broadcasted_iota(jnp.int32, (_B, _B), 0)
    col_ids = jax.lax.broadcasted_iota(jnp.int32, (_B, _B), 1)
    lane_ids = jax.lax.broadcasted_iota(jnp.int32, (1, _B), 1)

    def to_col(v_row):
        m = jnp.where(row_ids == col_ids, jnp.broadcast_to(v_row, (_B, _B)), 0.0)
        return jnp.sum(m, axis=1, keepdims=True)

    def outer(bi, carry):
        x1r = x1_ref[pl.ds(bi, 1), :]
        y1r = y1_ref[pl.ds(bi, 1), :]
        x2r = x2_ref[pl.ds(bi, 1), :]
        y2r = y2_ref[pl.ds(bi, 1), :]
        ar = area_ref[pl.ds(bi, 1), :]
        x1c = to_col(x1r)
        y1c = to_col(y1r)
        x2c = to_col(x2r)
        y2c = to_col(y2r)
        ac = to_col(ar)

        xx1 = jnp.maximum(x1c, x1r)
        yy1 = jnp.maximum(y1c, y1r)
        xx2 = jnp.minimum(x2c, x2r)
        yy2 = jnp.minimum(y2c, y2r)
        inter = jnp.maximum(xx2 - xx1, 0.0) * jnp.maximum(yy2 - yy1, 0.0)
        iou = inter / (ac + ar - inter)
        s_mat = jnp.where((iou > _IOU_T) & (col_ids > row_ids), 1, 0)

        kb = keep_ref[pl.ds(bi, 1), :]

        def intra(i, kb):
            keep_i = jnp.max(jnp.where(lane_ids == i, kb, 0))
            row_i = jnp.max(jnp.where(row_ids == i, s_mat, 0), axis=0, keepdims=True)
            return kb * (1 - row_i * keep_i)

        kb = jax.lax.fori_loop(0, _B, intra, kb)
        keep_ref[pl.ds(bi, 1), :] = kb
        kbc = to_col(kb.astype(jnp.float32))

        def inner(tb, carry2):
            tx1 = x1_ref[pl.ds(tb, 1), :]
            ty1 = y1_ref[pl.ds(tb, 1), :]
            tx2 = x2_ref[pl.ds(tb, 1), :]
            ty2 = y2_ref[pl.ds(tb, 1), :]
            ta = area_ref[pl.ds(tb, 1), :]
            xx1 = jnp.maximum(x1c, tx1)
            yy1 = jnp.maximum(y1c, ty1)
            xx2 = jnp.minimum(x2c, tx2)
            yy2 = jnp.minimum(y2c, ty2)
            inter = jnp.maximum(xx2 - xx1, 0.0) * jnp.maximum(yy2 - yy1, 0.0)
            iou = inter / (ac + ta - inter)
            sup = jnp.where((iou > _IOU_T) & (kbc > 0.5), 1, 0)
            sup_any = jnp.max(sup, axis=0, keepdims=True)
            keep_ref[pl.ds(tb, 1), :] = keep_ref[pl.ds(tb, 1), :] * (1 - sup_any)
            return carry2

        jax.lax.fori_loop(bi + 1, nb, inner, 0)
        return carry

    jax.lax.fori_loop(0, nb, outer, 0)


def _nms_keep_pallas(boxes_sorted):
    n = boxes_sorted.shape[0]
    nb = (n + _B - 1) // _B
    pad = nb * _B - n
    bp = jnp.concatenate(
        [boxes_sorted, jnp.zeros((pad, 4), boxes_sorted.dtype)], axis=0
    )
    x1 = bp[:, 0].reshape(nb, _B)
    y1 = bp[:, 1].reshape(nb, _B)
    x2 = bp[:, 2].reshape(nb, _B)
    y2 = bp[:, 3].reshape(nb, _B)
    keep = pl.pallas_call(
        _nms_kernel,
        out_shape=jax.ShapeDtypeStruct((nb, _B), jnp.int32),
        scratch_shapes=[pltpu.VMEM((nb, _B), jnp.float32)],
    )(x1, y1, x2, y2)
    return keep.reshape(-1)[:n].astype(bool)


def _smooth_l1_sum(pred, target):
    pred_b, target_b = jnp.broadcast_arrays(pred, target)
    d = jnp.abs(pred_b - target_b)
    loss = jnp.where(d < 1.0, 0.5 * d * d, d - 0.5)
    return jnp.sum(loss)


def _focal_loss(pred, target, alpha, gamma):
    logp = jax.nn.log_softmax(pred, axis=-1)
    ce = -jnp.sum(target * logp)
    p_t = jnp.exp(-ce)
    loss = alpha * (1.0 - p_t) ** gamma * ce
    return jnp.sum(loss)


def kernel(loc, conf, target_boxes, target_labels):
    alpha, gamma = 0.25, 2.0
    conf_threshold = 0.6
    confidence_scores = jnp.max(conf, axis=1)
    mask = confidence_scores > conf_threshold
    boxes_all = loc[0]
    score_all = conf[:, 0]
    order = jnp.argsort(-jnp.where(mask, score_all, -jnp.inf))
    boxes_sorted = boxes_all[order]
    keep_greedy = _nms_keep_pallas(boxes_sorted)
    keep_sorted = keep_greedy & mask[order]
    conf_sorted = conf[order]
    conf_eff = jnp.where(keep_sorted[:, None], conf_sorted, -jnp.inf)
    k = target_labels.shape[0]
    vals_t, idx_t = jax.lax.top_k(conf_eff.T, k)
    matched_conf_topk = vals_t.T
    indices = idx_t.T
    matched_conf_bin = (matched_conf_topk[:, 0] > 0.5).astype(jnp.float32)
    matched_target_labels = target_labels.astype(jnp.float32)
    pred_box = boxes_sorted[indices, :]
    loc_loss = _smooth_l1_sum(pred_box, target_boxes)
    conf_loss = _focal_loss(matched_conf_bin, matched_target_labels, alpha, gamma)
    compact_idx = jnp.cumsum(mask) - 1
    num_positives = jnp.sum(jnp.where(keep_sorted, compact_idx[order], 0)).astype(jnp.float32)
    return (loc_loss + conf_loss) / num_positives
